# Initial kernel scaffold; baseline (speedup 1.0000x reference)
import jax
import jax.numpy as jnp
from jax.experimental import pallas as pl


def kernel(x, edge_index, edge_attr, u, batch, W1a, b1a, W1b, b1b, W2a, b2a, W2b, b2b):
    raise NotImplementedError("write your pallas kernel here")


# speedup vs baseline: 1.0149x; 1.0149x over previous
import functools

import jax
import jax.numpy as jnp
from jax import lax
from jax.experimental import pallas as pl
from jax.experimental.pallas import tpu as pltpu
from jax.experimental.pallas import tpu_sc as plsc

_F32 = jnp.float32


def _tc_matmul(a, w, b=None, block_rows=2000):
    M, K = a.shape
    H = w.shape[1]
    grid = (M // block_rows,)

    def body(a_ref, w_ref, *rest):
        if b is not None:
            b_ref, out_ref = rest
            out_ref[...] = (
                jnp.dot(a_ref[...], w_ref[...], preferred_element_type=_F32)
                + b_ref[...]
            )
        else:
            (out_ref,) = rest
            out_ref[...] = jnp.dot(
                a_ref[...], w_ref[...], preferred_element_type=_F32
            )

    in_specs = [
        pl.BlockSpec((block_rows, K), lambda i: (i, 0)),
        pl.BlockSpec((K, H), lambda i: (0, 0)),
    ]
    args = [a, w]
    if b is not None:
        in_specs.append(pl.BlockSpec((1, H), lambda i: (0, 0)))
        args.append(b)
    return pl.pallas_call(
        body,
        grid=grid,
        in_specs=in_specs,
        out_specs=pl.BlockSpec((block_rows, H), lambda i: (i, 0)),
        out_shape=jax.ShapeDtypeStruct((M, H), _F32),
    )(*args)


def _sc_edge_aggregate(xa, eb, row, col):
    N, H = xa.shape
    E = row.shape[0]
    NC, NS = 2, 16
    NT = NC * NS
    EPT = E // NT
    B = 80
    CH = EPT // B
    ZR = 16
    NG = N // ZR
    QZ = (NG + NS - 1) // NS
    mesh = plsc.VectorSubcoreMesh(core_axis_name="c", subcore_axis_name="s")

    @functools.partial(
        pl.kernel,
        out_type=(
            jax.ShapeDtypeStruct((NC, N, H), _F32),
            jax.ShapeDtypeStruct((NC, N, 16), _F32),
        ),
        mesh=mesh,
        scratch_types=[
            pltpu.VMEM_SHARED((N, H), _F32),
            pltpu.VMEM_SHARED((N, 16), _F32),
            pltpu.VMEM((1, B), jnp.int32),
            pltpu.VMEM((1, B), jnp.int32),
            pltpu.VMEM((B, H), _F32),
            pltpu.VMEM((B, H), _F32),
            pltpu.VMEM((B, 16), _F32),
            pltpu.VMEM((ZR, H), _F32),
            pltpu.VMEM((ZR, 16), _F32),
            pltpu.SemaphoreType.DMA,
        ],
    )
    def sc_kernel(xa_hbm, eb_hbm, row_hbm, col_hbm, r_out, c_out,
                  r_sh, c_sh, row_v, col_v, g_v, e_v, ones_v, z_v, zc_v, sem):
        c = lax.axis_index("c")
        s = lax.axis_index("s")
        tid = c * NS + s

        @pl.loop(0, ZR)
        def _(i):
            for j in range(H // 16):
                z_v[i, pl.ds(j * 16, 16)] = jnp.zeros((16,), _F32)
            zc_v[i, pl.ds(0, 16)] = jnp.zeros((16,), _F32)

        @pl.loop(0, B)
        def _(i):
            ones_v[i, pl.ds(0, 16)] = jnp.ones((16,), _F32)

        @pl.loop(0, QZ)
        def _(q):
            g = jnp.minimum(s + q * NS, NG - 1)
            base = g * ZR
            pltpu.sync_copy(z_v, r_sh.at[pl.ds(base, ZR)])
            pltpu.sync_copy(zc_v, c_sh.at[pl.ds(base, ZR)])

        plsc.subcore_barrier()

        @pl.loop(0, CH)
        def _(k):
            ebase = tid * EPT + k * B
            pltpu.sync_copy(row_hbm.at[pl.ds(ebase, B)], row_v.at[0])
            pltpu.sync_copy(col_hbm.at[pl.ds(ebase, B)], col_v.at[0])
            pltpu.async_copy(xa_hbm.at[row_v.at[0]], g_v, sem).wait()
            pltpu.sync_copy(eb_hbm.at[pl.ds(ebase, B)], e_v)

            @pl.loop(0, B)
            def _(i):
                for j in range(H // 16):
                    sl = pl.ds(j * 16, 16)
                    g_v[i, sl] = jnp.maximum(g_v[i, sl] + e_v[i, sl], 0.0)

            pltpu.sync_copy(g_v, r_sh.at[col_v.at[0]], add=True)
            pltpu.sync_copy(ones_v, c_sh.at[col_v.at[0]], add=True)

        plsc.subcore_barrier()

        def copy_out(base):
            pltpu.sync_copy(r_sh.at[pl.ds(base, ZR)], g_v.at[pl.ds(0, ZR)])
            pltpu.sync_copy(g_v.at[pl.ds(0, ZR)],
                            r_out.at[c, pl.ds(base, ZR)])
            pltpu.sync_copy(c_sh.at[pl.ds(base, ZR)],
                            ones_v.at[pl.ds(0, ZR)])
            pltpu.sync_copy(ones_v.at[pl.ds(0, ZR)],
                            c_out.at[c, pl.ds(base, ZR)])

        @pl.loop(0, QZ)
        def _(q):
            copy_out(jnp.minimum(s + q * NS, NG - 1) * ZR)

    return sc_kernel(xa, eb, row, col)


def _tc_node_mlp(x, rp, cp, u, batch2d, W1b, b1b, W2a_x, W2a_m, W2a_u,
                 b2a, W2b, b2b, block_rows=2000):
    N, NF = x.shape
    H = W1b.shape[0]
    G = u.shape[0]
    grid = (N // block_rows,)
    BN = block_rows

    def body(x_ref, r_ref, c_ref, b_ref, u_ref, w1b_ref, b1b_ref, w2ax_ref,
             w2am_ref, w2au_ref, b2a_ref, w2b_ref, b2b_ref, out_ref):
        R = r_ref[0] + r_ref[1]
        cnt = c_ref[0, :, 0:1] + c_ref[1, :, 0:1]
        meanR = R / jnp.maximum(cnt, 1.0)
        pos = jnp.where(cnt > 0.0, 1.0, 0.0)
        mh = (
            jnp.dot(meanR, w1b_ref[...], preferred_element_type=_F32)
            + pos * b1b_ref[...]
        )
        ug = jnp.dot(u_ref[...], w2au_ref[...], preferred_element_type=_F32)
        oh = (b_ref[...] == lax.broadcasted_iota(jnp.int32, (BN, G), 1))
        z = (
            jnp.dot(x_ref[...], w2ax_ref[...], preferred_element_type=_F32)
            + jnp.dot(mh, w2am_ref[...], preferred_element_type=_F32)
            + jnp.dot(oh.astype(_F32), ug, preferred_element_type=_F32)
            + b2a_ref[...]
        )
        out_ref[...] = (
            jnp.dot(jnp.maximum(z, 0.0), w2b_ref[...],
                    preferred_element_type=_F32)
            + b2b_ref[...]
        )

    full = lambda shape: pl.BlockSpec(shape, lambda i: tuple(0 for _ in shape))
    return pl.pallas_call(
        body,
        grid=grid,
        in_specs=[
            pl.BlockSpec((BN, NF), lambda i: (i, 0)),
            pl.BlockSpec((2, BN, H), lambda i: (0, i, 0)),
            pl.BlockSpec((2, BN, 16), lambda i: (0, i, 0)),
            pl.BlockSpec((BN, 1), lambda i: (i, 0)),
            full((G, u.shape[1])),
            full((H, H)),
            full((1, H)),
            full((NF, H)),
            full((H, H)),
            full((u.shape[1], H)),
            full((1, H)),
            full((H, NF)),
            full((1, NF)),
        ],
        out_specs=pl.BlockSpec((BN, NF), lambda i: (i, 0)),
        out_shape=jax.ShapeDtypeStruct((N, NF), _F32),
    )(x, rp, cp, batch2d, u, W1b, b1b.reshape(1, -1), W2a_x, W2a_m, W2a_u,
      b2a.reshape(1, -1), W2b, b2b.reshape(1, -1))


def kernel(x, edge_index, edge_attr, u, batch, W1a, b1a, W1b, b1b, W2a, b2a,
           W2b, b2b):
    N, NF = x.shape
    H = W1b.shape[0]
    row = edge_index[0]
    col = edge_index[1]

    xa = _tc_matmul(x, W1a[:NF])
    eb = _tc_matmul(edge_attr, W1a[NF:], b1a.reshape(1, -1))
    r = jnp.maximum(jnp.take(xa, row, axis=0) + eb, 0.0)
    half = r.shape[0] // 2
    rp = jnp.stack([
        jax.ops.segment_sum(r[:half], col[:half], num_segments=N),
        jax.ops.segment_sum(r[half:], col[half:], num_segments=N)])
    onesE = jnp.ones((half, 16), _F32)
    cp = jnp.stack([
        jax.ops.segment_sum(onesE, col[:half], num_segments=N),
        jax.ops.segment_sum(onesE, col[half:], num_segments=N)])
    return _tc_node_mlp(
        x, rp, cp, u, batch.reshape(N, 1), W1b, b1b,
        W2a[:NF], W2a[NF:NF + H], W2a[NF + H:], b2a, W2b, b2b,
    )
